# baseline (speedup 1.0000x reference)
import functools

import jax
import jax.numpy as jnp
from jax.experimental import pallas as pl
from jax.experimental.pallas import tpu as pltpu
from jax.experimental.pallas import tpu_sc as plsc

BATCH = 4096
SEQ = 200
EMBED = 64
LANES = 16
BTILE = 128
NBT = BATCH // BTILE
NWIN = SEQ * NBT
NWORK = 32
STEPS = NWIN // NWORK


def _prep_body(seg_table_ref, pos_table_ref, comb_ref):
    pos = pos_table_ref[...]
    seg = seg_table_ref[...]
    comb_ref[...] = pos + seg[:, None, :]


_prep = pl.pallas_call(
    _prep_body,
    out_shape=jax.ShapeDtypeStruct((3, SEQ, EMBED), jnp.float32),
)


def _embed_kernel(tokp_hbm, comb_hbm, seqt_hbm, segt_hbm, out_hbm,
                  comb_tbl,
                  sq0, sq1, sg0, sg1, si0, si1,
                  t0, t1, ob0, ob1,
                  sem_i0, sem_i1, sem_g0, sem_g1, sem_o0, sem_o1):
    pltpu.sync_copy(comb_hbm, comb_tbl)

    wid = jax.lax.axis_index("subcore") * 2 + jax.lax.axis_index("core")
    base = wid * STEPS

    iota = jax.lax.iota(jnp.int32, LANES)

    slots = (
        (sq0, sg0, si0, t0, ob0, sem_i0, sem_g0, sem_o0),
        (sq1, sg1, si1, t1, ob1, sem_i1, sem_g1, sem_o1),
    )

    def win_coords(k):
        w = base + k
        return w // NBT, w % NBT

    def issue_idx(k, slot):
        sq, sg, _, _, _, sem_i, _, _ = slot
        s, bt = win_coords(k)
        b0 = bt * BTILE
        pltpu.async_copy(seqt_hbm.at[s, pl.ds(b0, BTILE)], sq, sem_i)
        pltpu.async_copy(segt_hbm.at[s, pl.ds(b0, BTILE)], sg, sem_i)

    def wait_idx(slot):
        sq, sg, _, _, _, sem_i, _, _ = slot
        pltpu.make_async_copy(seqt_hbm.at[0, pl.ds(0, BTILE)], sq, sem_i).wait()
        pltpu.make_async_copy(segt_hbm.at[0, pl.ds(0, BTILE)], sg, sem_i).wait()

    def fill_pair_idx_and_gather(slot):
        sq, _, si, t, _, _, sem_g, _ = slot
        for g in range(BTILE // LANES):
            sl = pl.ds(g * LANES, LANES)
            si[sl] = jax.lax.shift_right_logical(sq[sl], 1)
        pltpu.async_copy(tokp_hbm.at[si], t, sem_g)

    def wait_gather(slot):
        _, _, si, t, _, _, sem_g, _ = slot
        pltpu.make_async_copy(tokp_hbm.at[si], t, sem_g).wait()

    def out_slice(k):
        s, bt = win_coords(k)
        return out_hbm.at[s, :, bt]

    def wait_out(k, slot):
        pltpu.make_async_copy(ob := slot[4], out_slice(k), slot[7]).wait()

    for kp in range(2):
        sq, sg, _, _, _, _, _, _ = slots[kp]
        s, bt = win_coords(kp)
        b0 = bt * BTILE
        pltpu.sync_copy(seqt_hbm.at[s, pl.ds(b0, BTILE)], sq)
        pltpu.sync_copy(segt_hbm.at[s, pl.ds(b0, BTILE)], sg)
        fill_pair_idx_and_gather(slots[kp])

    def stage(k, si_slot):
        slot = slots[si_slot]
        sq, sg, si, t, ob, sem_i, sem_g, sem_o = slot
        s, bt = win_coords(k)
        wait_gather(slot)

        @pl.when(k >= 2)
        def _():
            pltpu.make_async_copy(ob, out_slice(k), sem_o).wait()

        s64 = s * EMBED

        @pl.loop(0, BTILE // LANES)
        def _(g):
            gsl = pl.ds(g * LANES, LANES)
            sq_v = sq[gsl]
            h64 = jax.lax.shift_left(
                jax.lax.bitwise_and(sq_v, 1), 6)
            rows = g * LANES + iota
            crow = sg[gsl] * SEQ + s
            gofs = pl.ds(g * LANES, LANES)
            for d in range(EMBED):
                tval = plsc.load_gather(t, [rows, h64 + d])
                cval = plsc.load_gather(comb_tbl, [crow, iota * 0 + d])
                ob[d // 8, d % 8, gofs] = tval + cval

        pltpu.async_copy(ob, out_slice(k), sem_o)

        @pl.when(k + 2 < STEPS)
        def _():
            issue_idx(k + 2, slot)
            wait_idx(slot)
            fill_pair_idx_and_gather(slot)

    @pl.loop(0, STEPS, step=2)
    def _(k):
        stage(k, 0)
        stage(k + 1, 1)

    wait_out(STEPS - 2, slots[0])
    wait_out(STEPS - 1, slots[1])


def _make_embed():
    mesh = plsc.VectorSubcoreMesh(
        core_axis_name="core", subcore_axis_name="subcore"
    )
    return pl.kernel(
        _embed_kernel,
        out_type=jax.ShapeDtypeStruct(
            (SEQ, EMBED // 8, NBT, 8, BTILE), jnp.float32),
        mesh=mesh,
        compiler_params=pltpu.CompilerParams(
            use_tc_tiling_on_sc=False, needs_layout_passes=False),
        scratch_types=[
            pltpu.VMEM((3 * SEQ, EMBED), jnp.float32),
            pltpu.VMEM((BTILE,), jnp.int32),
            pltpu.VMEM((BTILE,), jnp.int32),
            pltpu.VMEM((BTILE,), jnp.int32),
            pltpu.VMEM((BTILE,), jnp.int32),
            pltpu.VMEM((BTILE,), jnp.int32),
            pltpu.VMEM((BTILE,), jnp.int32),
            pltpu.VMEM((BTILE, BTILE), jnp.float32),
            pltpu.VMEM((BTILE, BTILE), jnp.float32),
            pltpu.VMEM((EMBED // 8, 8, BTILE), jnp.float32),
            pltpu.VMEM((EMBED // 8, 8, BTILE), jnp.float32),
        ] + [pltpu.SemaphoreType.DMA] * 6,
    )


_embed = _make_embed()


@jax.jit
def kernel(sequence, segment_label, token_table, segment_table, pos_table):
    comb = _prep(segment_table, pos_table)
    out5 = _embed(
        token_table.reshape(500000, BTILE),
        comb.reshape(3 * SEQ, EMBED),
        sequence.astype(jnp.int32).T,
        segment_label.astype(jnp.int32).T,
    )
    return out5.transpose(2, 4, 0, 1, 3).reshape(BATCH, SEQ, EMBED)


# speedup vs baseline: 2.1357x; 2.1357x over previous
import functools

import jax
import jax.numpy as jnp
from jax.experimental import pallas as pl
from jax.experimental.pallas import tpu as pltpu
from jax.experimental.pallas import tpu_sc as plsc

BATCH = 4096
SEQ = 200
EMBED = 64
LANES = 16
BTILE = 128
NBT = BATCH // BTILE
NWIN = SEQ * NBT
NWORK = 32
STEPS = NWIN // NWORK


def _prep_body(seg_table_ref, pos_table_ref, comb_ref):
    pos = pos_table_ref[...]
    seg = seg_table_ref[...]
    comb_ref[...] = pos + seg[:, None, :]


_prep = pl.pallas_call(
    _prep_body,
    out_shape=jax.ShapeDtypeStruct((3, SEQ, EMBED), jnp.float32),
)


def _embed_kernel(tok_hbm, comb_hbm, seqt_hbm, segt_hbm, out_hbm,
                  comb_tbl,
                  sq0, sq1, sg0, sg1, cr0, cr1,
                  t0, t1, ob0, ob1,
                  sem_i0, sem_i1, sem_g0, sem_g1, sem_o0, sem_o1):
    pltpu.sync_copy(comb_hbm, comb_tbl)

    wid = jax.lax.axis_index("subcore") * 2 + jax.lax.axis_index("core")
    base = wid * STEPS

    iota = jax.lax.iota(jnp.int32, LANES)

    slots = (
        (sq0, sg0, cr0, t0, ob0, sem_i0, sem_g0, sem_o0),
        (sq1, sg1, cr1, t1, ob1, sem_i1, sem_g1, sem_o1),
    )

    def win_coords(k):
        w = base + k
        return w // NBT, w % NBT

    def issue_idx(k, slot):
        sq, sg, _, _, _, sem_i, _, _ = slot
        s, bt = win_coords(k)
        b0 = bt * BTILE
        pltpu.async_copy(seqt_hbm.at[s, pl.ds(b0, BTILE)], sq, sem_i)
        pltpu.async_copy(segt_hbm.at[s, pl.ds(b0, BTILE)], sg, sem_i)

    def wait_idx(slot):
        sq, sg, _, _, _, sem_i, _, _ = slot
        pltpu.make_async_copy(seqt_hbm.at[0, pl.ds(0, BTILE)], sq, sem_i).wait()
        pltpu.make_async_copy(segt_hbm.at[0, pl.ds(0, BTILE)], sg, sem_i).wait()

    def issue_gather(slot):
        sq, _, _, t, _, _, sem_g, _ = slot
        pltpu.async_copy(tok_hbm.at[sq], t, sem_g)

    def wait_gather(slot):
        sq, _, _, t, _, _, sem_g, _ = slot
        pltpu.make_async_copy(tok_hbm.at[sq], t, sem_g).wait()

    def out_slice(k):
        s, bt = win_coords(k)
        return out_hbm.at[s, :, bt]

    def wait_out(k, slot):
        pltpu.make_async_copy(slot[4], out_slice(k), slot[7]).wait()

    for kp in range(2):
        sq, sg, _, _, _, _, _, _ = slots[kp]
        s, bt = win_coords(kp)
        b0 = bt * BTILE
        pltpu.sync_copy(seqt_hbm.at[s, pl.ds(b0, BTILE)], sq)
        pltpu.sync_copy(segt_hbm.at[s, pl.ds(b0, BTILE)], sg)
        issue_gather(slots[kp])

    def stage(k, slot_i):
        slot = slots[slot_i]
        sq, sg, cr, t, ob, sem_i, sem_g, sem_o = slot
        s, bt = win_coords(k)
        wait_gather(slot)

        @pl.when(k >= 2)
        def _():
            pltpu.make_async_copy(ob, out_slice(k), sem_o).wait()

        for g in range(BTILE // LANES):
            gsl = pl.ds(g * LANES, LANES)
            cr[gsl] = sg[gsl] * SEQ + s

        @pl.loop(0, LANES)
        def _(j):
            jm = jax.lax.bitwise_and(iota + j, LANES - 1)
            for r0 in range(BTILE // LANES):
                r_v = r0 * LANES + iota
                crow = cr[pl.ds(r0 * LANES, LANES)]
                for c0 in range(EMBED // LANES):
                    c_v = c0 * LANES + jm
                    tv = plsc.load_gather(t, [r_v, c_v])
                    cv = plsc.load_gather(comb_tbl, [crow, c_v])
                    plsc.store_scatter(
                        ob,
                        [jax.lax.shift_right_logical(c_v, 3),
                         jax.lax.bitwise_and(c_v, 7),
                         r_v],
                        tv + cv,
                    )

        pltpu.async_copy(ob, out_slice(k), sem_o)

        @pl.when(k + 2 < STEPS)
        def _():
            issue_idx(k + 2, slot)
            wait_idx(slot)
            issue_gather(slot)

    @pl.loop(0, STEPS, step=2)
    def _(k):
        stage(k, 0)
        stage(k + 1, 1)

    wait_out(STEPS - 2, slots[0])
    wait_out(STEPS - 1, slots[1])


def _make_embed():
    mesh = plsc.VectorSubcoreMesh(
        core_axis_name="core", subcore_axis_name="subcore"
    )
    return pl.kernel(
        _embed_kernel,
        out_type=jax.ShapeDtypeStruct(
            (SEQ, EMBED // 8, NBT, 8, BTILE), jnp.float32),
        mesh=mesh,
        compiler_params=pltpu.CompilerParams(
            use_tc_tiling_on_sc=False, needs_layout_passes=False),
        scratch_types=[
            pltpu.VMEM((3 * SEQ, EMBED), jnp.float32),
            pltpu.VMEM((BTILE,), jnp.int32),
            pltpu.VMEM((BTILE,), jnp.int32),
            pltpu.VMEM((BTILE,), jnp.int32),
            pltpu.VMEM((BTILE,), jnp.int32),
            pltpu.VMEM((BTILE,), jnp.int32),
            pltpu.VMEM((BTILE,), jnp.int32),
            pltpu.VMEM((BTILE, EMBED), jnp.float32),
            pltpu.VMEM((BTILE, EMBED), jnp.float32),
            pltpu.VMEM((EMBED // 8, 8, BTILE), jnp.float32),
            pltpu.VMEM((EMBED // 8, 8, BTILE), jnp.float32),
        ] + [pltpu.SemaphoreType.DMA] * 6,
    )


_embed = _make_embed()


@jax.jit
def kernel(sequence, segment_label, token_table, segment_table, pos_table):
    comb = _prep(segment_table, pos_table)
    out5 = _embed(
        token_table,
        comb.reshape(3 * SEQ, EMBED),
        sequence.astype(jnp.int32).T,
        segment_label.astype(jnp.int32).T,
    )
    return out5.transpose(2, 4, 0, 1, 3).reshape(BATCH, SEQ, EMBED)
